# baseline (speedup 1.0000x reference)
import math

import jax
import jax.numpy as jnp
from jax import lax
from jax.experimental import pallas as pl
from jax.experimental.pallas import tpu as pltpu
from jax.experimental.pallas import tpu_sc as plsc

_NUM_CORES = 2
_NUM_SUBCORES = 16
_NUM_WORKERS = _NUM_CORES * _NUM_SUBCORES
_LANES = 16

_IDX_W = 128
_K = 2
_CHUNK = _K * _IDX_W


def _emb_kernel_body(nchunks, dim, x_hbm, lut_hbm, out_hbm,
                     ib0, ib1, gb0, gb1, ob0, ob1,
                     isem0, isem1, gsem0, gsem1, osem0, osem1):
  scale = math.sqrt(dim)
  nslice = dim // _LANES
  rows_per_w = nchunks * _CHUNK
  irows_per_w = nchunks * _K

  wid = lax.axis_index("s") * _NUM_CORES + lax.axis_index("c")
  irow0 = wid * irows_per_w
  row0 = wid * rows_per_w

  ibufs = (ib0, ib1)
  gbufs = (gb0, gb1)
  obufs = (ob0, ob1)
  isems = (isem0, isem1)
  gsems = (gsem0, gsem1)
  osems = (osem0, osem1)

  def idx_start(g, b):
    pltpu.async_copy(x_hbm.at[pl.ds(irow0 + g * _K, _K)], ibufs[b], isems[b])

  def idx_wait(g, b):
    pltpu.make_async_copy(
        x_hbm.at[pl.ds(irow0 + g * _K, _K)], ibufs[b], isems[b]).wait()

  def gather_start(b):
    for j in range(_K):
      pltpu.async_copy(
          lut_hbm.at[ibufs[b].at[j]],
          gbufs[b].at[pl.ds(j * _IDX_W, _IDX_W)],
          gsems[b])

  def gather_wait(b):
    for j in range(_K):
      pltpu.make_async_copy(
          lut_hbm.at[ibufs[b].at[j]],
          gbufs[b].at[pl.ds(j * _IDX_W, _IDX_W)],
          gsems[b]).wait()

  def out_start(g, b):
    pltpu.async_copy(
        obufs[b], out_hbm.at[pl.ds(row0 + g * _CHUNK, _CHUNK)], osems[b])

  def out_wait(g, b):
    pltpu.make_async_copy(
        obufs[b], out_hbm.at[pl.ds(row0 + g * _CHUNK, _CHUNK)], osems[b]).wait()

  for b in range(2):
    idx_start(b, b)
  for b in range(2):
    idx_wait(b, b)
    gather_start(b)

  @pl.loop(0, nchunks, step=2)
  def _steady(g0):
    for b in range(2):
      g = g0 + b
      gather_wait(b)

      @pl.when(g + 2 < nchunks)
      def _():
        idx_start(g + 2, b)

      @pl.when(g >= 2)
      def _():
        out_wait(g - 2, b)

      gbuf = gbufs[b]
      obuf = obufs[b]

      @plsc.parallel_loop(0, _CHUNK, unroll=8)
      def _scale(i):
        for j in range(nslice):
          sl = pl.ds(j * _LANES, _LANES)
          obuf[i, sl] = gbuf[i, sl] * scale

      @pl.when(g + 2 < nchunks)
      def _():
        idx_wait(g + 2, b)
        gather_start(b)

      out_start(g, b)

  for b in range(2):
    out_wait(nchunks - 2 + b, b)


def kernel(x, lut):
  batch_shape = x.shape
  dim = lut.shape[1]
  n = x.size
  assert n % (_NUM_WORKERS * _CHUNK) == 0
  assert dim % _LANES == 0
  nchunks = n // (_NUM_WORKERS * _CHUNK)

  x2d = x.reshape(-1).astype(jnp.int32).reshape(n // _IDX_W, _IDX_W)

  mesh = plsc.VectorSubcoreMesh(
      core_axis_name="c", subcore_axis_name="s",
      num_cores=_NUM_CORES, num_subcores=_NUM_SUBCORES)

  run = pl.kernel(
      lambda *refs: _emb_kernel_body(nchunks, dim, *refs),
      out_type=jax.ShapeDtypeStruct((n, dim), jnp.float32),
      mesh=mesh,
      scratch_types=[
          pltpu.VMEM((_K, _IDX_W), jnp.int32),
          pltpu.VMEM((_K, _IDX_W), jnp.int32),
          pltpu.VMEM((_CHUNK, dim), jnp.float32),
          pltpu.VMEM((_CHUNK, dim), jnp.float32),
          pltpu.VMEM((_CHUNK, dim), jnp.float32),
          pltpu.VMEM((_CHUNK, dim), jnp.float32),
          pltpu.SemaphoreType.DMA,
          pltpu.SemaphoreType.DMA,
          pltpu.SemaphoreType.DMA,
          pltpu.SemaphoreType.DMA,
          pltpu.SemaphoreType.DMA,
          pltpu.SemaphoreType.DMA,
      ],
      compiler_params=pltpu.CompilerParams(use_tc_tiling_on_sc=False),
      name="sc_embedding_lookup",
  )
  out = run(x2d, lut)
  return out.reshape(*batch_shape, dim)
